# baseline (speedup 1.0000x reference)
import jax
import jax.numpy as jnp
from jax.experimental import pallas as pl
from jax.experimental.pallas import tpu as pltpu

IMAGE_SIZE = 512.0
CONFIDENCE_THRESHOLD = 0.01
NMS_THRESHOLD = 0.45
MAX_PER_IMAGE = 100
CENTER_VARIANCE = 0.1
SIZE_VARIANCE = 0.2
NEG = -1e30
CLASS_OFFSET = 4.0 * IMAGE_SIZE

N_PRIORS = 20000
N_CLASSES = 21
C = N_CLASSES - 1
N_BLOCKS = 157
N_PAD = N_BLOCKS * 128
N_BM = 256
N_KEEP_PAD = 128


def _nms_kernel(logits_ref, loc_ref, prior_ref, out_ref, cand_ref, cm_ref):
    logits = logits_ref[...]
    m = jnp.max(logits, axis=0, keepdims=True)
    e = jnp.exp(logits - m)
    probs = e / jnp.sum(e, axis=0, keepdims=True)
    scores = probs[1:, :]

    loc = loc_ref[...]
    pr = prior_ref[...]
    cx = loc[0:1, :] * CENTER_VARIANCE * pr[2:3, :] + pr[0:1, :]
    cy = loc[1:2, :] * CENTER_VARIANCE * pr[3:4, :] + pr[1:2, :]
    w = jnp.exp(loc[2:3, :] * SIZE_VARIANCE) * pr[2:3, :]
    h = jnp.exp(loc[3:4, :] * SIZE_VARIANCE) * pr[3:4, :]
    x1 = (cx - w * 0.5) * IMAGE_SIZE
    y1 = (cy - h * 0.5) * IMAGE_SIZE
    x2 = (cx + w * 0.5) * IMAGE_SIZE
    y2 = (cy + h * 0.5) * IMAGE_SIZE

    S = jnp.where(scores > CONFIDENCE_THRESHOLD, scores, NEG)

    comb = jnp.concatenate(
        [S, x1, y1, x2, y2, jnp.full((8, N_PRIORS), NEG, jnp.float32)], axis=0)
    cand_ref[...] = jnp.transpose(comb)

    colmax_flat = jnp.max(S, axis=0, keepdims=True)
    colmax_flat = jnp.concatenate(
        [colmax_flat, jnp.full((1, N_PAD - N_PRIORS), NEG, jnp.float32)],
        axis=1)
    cm0 = jnp.reshape(colmax_flat, (N_BLOCKS, 128))
    cm_ref[...] = cm0

    rowmax0 = jnp.max(cm0, axis=1, keepdims=True)
    bm_t = jnp.transpose(rowmax0)
    bm0 = jnp.concatenate(
        [bm_t, jnp.full((1, N_BM - N_BLOCKS), NEG, jnp.float32)], axis=1)

    lane32 = jax.lax.broadcasted_iota(
        jnp.int32, (1, 32), 1).astype(jnp.float32)
    lane128 = jax.lax.broadcasted_iota(
        jnp.int32, (1, N_KEEP_PAD), 1).astype(jnp.float32)
    blane = jax.lax.broadcasted_iota(
        jnp.int32, (1, N_BM), 1).astype(jnp.float32)
    sub8 = jax.lax.broadcasted_iota(jnp.int32, (8, N_KEEP_PAD), 0)
    is_score = lane32 < float(C)

    zero8 = jnp.zeros((8, N_KEEP_PAD), jnp.float32)
    zerolane = jnp.zeros((1, N_KEEP_PAD), jnp.float32)
    nextoh0 = jnp.where(lane128 == 0.0, 1.0, 0.0)
    oh_last = jnp.where(lane128 == float(MAX_PER_IMAGE - 1), 1.0, 0.0)

    def _find_block(bm, gm):
        return jnp.min(jnp.where(bm == gm, blane, 1e9),
                       axis=1, keepdims=True)

    def _find_lane(cmrow, gm):
        return jnp.minimum(
            jnp.min(jnp.where(cmrow == gm, lane128, 1e9),
                    axis=1, keepdims=True), 127.0)

    def cond(carry):
        return carry[0] > -0.5

    def body(carry):
        p_code, bm, kept, valid, nextoh, outv = carry
        p = p_code.astype(jnp.int32)
        pb_f = jnp.floor(p_code * (1.0 / 128.0))
        lane_f = p_code - pb_f * 128.0
        pb = pb_f.astype(jnp.int32)

        row = cand_ref[pl.ds(p, 1), :]
        rs = jnp.where(is_score, row, NEG)
        gmax = jnp.max(rs, axis=1, keepdims=True)
        r_v = jnp.min(jnp.where(rs == gmax, lane32, 1e9),
                      axis=1, keepdims=True)

        bx1 = row[:, C:C + 1]
        by1 = row[:, C + 1:C + 2]
        bx2 = row[:, C + 2:C + 3]
        by2 = row[:, C + 3:C + 4]

        off = (r_v + 1.0) * CLASS_OFFSET
        bxo1 = bx1 + off
        byo1 = by1 + off
        bxo2 = bx2 + off
        byo2 = by2 + off
        barea = (bxo2 - bxo1) * (byo2 - byo1)

        kox1 = kept[0:1, :]
        koy1 = kept[1:2, :]
        kox2 = kept[2:3, :]
        koy2 = kept[3:4, :]
        karea = kept[4:5, :]
        iw = jnp.maximum(jnp.minimum(kox2, bxo2) - jnp.maximum(kox1, bxo1),
                         0.0)
        ih = jnp.maximum(jnp.minimum(koy2, byo2) - jnp.maximum(koy1, byo1),
                         0.0)
        inter = iw * ih
        iou = inter / (karea + barea - inter + 1e-9)
        hit = (iou > NMS_THRESHOLD) & (valid > 0.5)
        suppf = jnp.max(jnp.where(hit, 1.0, 0.0),
                        axis=1, keepdims=True)

        at_lane = nextoh * (1.0 - suppf)
        at_mask = at_lane > 0.5
        kval = jnp.where(sub8 == 0, bxo1,
                         jnp.where(sub8 == 1, byo1,
                                   jnp.where(sub8 == 2, bxo2,
                                             jnp.where(sub8 == 3, byo2,
                                                       barea))))
        kept_new = jnp.where(at_mask & (sub8 < 5), kval, kept)

        oval = jnp.where(sub8 == 0, gmax,
                         jnp.where(sub8 == 1, r_v + 1.0,
                                   jnp.where(sub8 == 2, bx1,
                                             jnp.where(sub8 == 3, by1,
                                                       jnp.where(sub8 == 4,
                                                                 bx2, by2)))))
        outv_new = jnp.where(at_mask & (sub8 < 6), oval, outv)

        valid_new = valid + at_lane
        shifted = jnp.concatenate([jnp.zeros((1, 1), jnp.float32),
                                   nextoh[:, :N_KEEP_PAD - 1]], axis=1)
        nextoh_new = shifted * (1.0 - suppf) + nextoh * suppf

        row_new = jnp.where(lane32 == r_v, NEG, row)
        cand_ref[pl.ds(p, 1), :] = row_new
        head = jnp.max(jnp.where(is_score, row_new, NEG),
                       axis=1, keepdims=True)
        cmrow = cm_ref[pl.ds(pb, 1), :]
        cmrow_new = jnp.where(lane128 == lane_f, head, cmrow)
        cm_ref[pl.ds(pb, 1), :] = cmrow_new
        rowmax = jnp.max(cmrow_new, axis=1, keepdims=True)
        bm_new = jnp.where(blane == pb_f, rowmax, bm)

        gmax_new = jnp.max(bm_new, axis=1, keepdims=True)
        b_code = _find_block(bm_new, gmax_new)
        b = b_code[0, 0].astype(jnp.int32)
        cmrow2 = cm_ref[pl.ds(b, 1), :]
        l_code = _find_lane(cmrow2, gmax_new)
        p_next = b_code * 128.0 + l_code

        full = jnp.max(valid_new * oh_last, axis=1, keepdims=True)
        stop = (full > 0.5) | (gmax_new < -1e20)
        p_code_next = jnp.where(stop, -1.0, p_next)[0, 0]

        return (p_code_next, bm_new, kept_new, valid_new, nextoh_new,
                outv_new)

    gm0 = jnp.max(bm0, axis=1, keepdims=True)
    b0_code = _find_block(bm0, gm0)
    b0 = b0_code[0, 0].astype(jnp.int32)
    cmrow_init = cm_ref[pl.ds(b0, 1), :]
    p0 = b0_code * 128.0 + _find_lane(cmrow_init, gm0)
    p_code0 = jnp.where(gm0 < -1e20, -1.0, p0)[0, 0]

    final = jax.lax.while_loop(
        cond, body, (p_code0, bm0, zero8, zerolane, nextoh0, zero8))
    out_ref[...] = final[5]


@jax.jit
def kernel(cls_logits, bbox_pred, priors):
    logits_t = jnp.transpose(cls_logits[0])
    loc_t = jnp.transpose(bbox_pred[0])
    prior_t = jnp.transpose(priors)

    out = pl.pallas_call(
        _nms_kernel,
        out_shape=jax.ShapeDtypeStruct((8, N_KEEP_PAD), jnp.float32),
        scratch_shapes=[pltpu.VMEM((N_PRIORS, 32), jnp.float32),
                        pltpu.VMEM((N_BLOCKS, 128), jnp.float32)],
    )(logits_t, loc_t, prior_t)

    out_scores = out[0, :MAX_PER_IMAGE]
    out_labels = out[1, :MAX_PER_IMAGE].astype(jnp.int32)
    out_boxes = jnp.stack(
        [out[2, :MAX_PER_IMAGE], out[3, :MAX_PER_IMAGE],
         out[4, :MAX_PER_IMAGE], out[5, :MAX_PER_IMAGE]], axis=-1)
    return out_boxes, out_scores, out_labels


# speedup vs baseline: 1.0692x; 1.0692x over previous
import jax
import jax.numpy as jnp
from jax.experimental import pallas as pl
from jax.experimental.pallas import tpu as pltpu

IMAGE_SIZE = 512.0
CONFIDENCE_THRESHOLD = 0.01
NMS_THRESHOLD = 0.45
MAX_PER_IMAGE = 100
CENTER_VARIANCE = 0.1
SIZE_VARIANCE = 0.2
NEG = -1e30
CLASS_OFFSET = 4.0 * IMAGE_SIZE

N_PRIORS = 20000
N_CLASSES = 21
C = N_CLASSES - 1
N_BLOCKS = 157
N_PAD = N_BLOCKS * 128
N_KEEP_PAD = 128
BIG_I = 2**30


def _nms_kernel(logits_ref, loc_ref, prior_ref, out_ref, cand_ref):
    logits = logits_ref[...]
    m = jnp.max(logits, axis=0, keepdims=True)
    e = jnp.exp(logits - m)
    probs = e / jnp.sum(e, axis=0, keepdims=True)
    scores = probs[1:, :]

    loc = loc_ref[...]
    pr = prior_ref[...]
    cx = loc[0:1, :] * CENTER_VARIANCE * pr[2:3, :] + pr[0:1, :]
    cy = loc[1:2, :] * CENTER_VARIANCE * pr[3:4, :] + pr[1:2, :]
    w = jnp.exp(loc[2:3, :] * SIZE_VARIANCE) * pr[2:3, :]
    h = jnp.exp(loc[3:4, :] * SIZE_VARIANCE) * pr[3:4, :]
    x1 = (cx - w * 0.5) * IMAGE_SIZE
    y1 = (cy - h * 0.5) * IMAGE_SIZE
    x2 = (cx + w * 0.5) * IMAGE_SIZE
    y2 = (cy + h * 0.5) * IMAGE_SIZE

    S = jnp.where(scores > CONFIDENCE_THRESHOLD, scores, NEG)

    comb = jnp.concatenate(
        [S, x1, y1, x2, y2, jnp.full((8, N_PRIORS), NEG, jnp.float32)], axis=0)
    cand_ref[...] = jnp.transpose(comb)

    colmax_flat = jnp.max(S, axis=0, keepdims=True)
    colmax_flat = jnp.concatenate(
        [colmax_flat, jnp.full((1, N_PAD - N_PRIORS), NEG, jnp.float32)],
        axis=1)
    cm0 = jnp.reshape(colmax_flat, (N_BLOCKS, 128))

    lane32 = jax.lax.broadcasted_iota(
        jnp.int32, (1, 32), 1).astype(jnp.float32)
    lane128 = jax.lax.broadcasted_iota(
        jnp.int32, (1, N_KEEP_PAD), 1).astype(jnp.float32)
    sub8 = jax.lax.broadcasted_iota(jnp.int32, (8, N_KEEP_PAD), 0)
    fidx = (jax.lax.broadcasted_iota(jnp.int32, (N_BLOCKS, 128), 0) * 128 +
            jax.lax.broadcasted_iota(jnp.int32, (N_BLOCKS, 128), 1)
            ).astype(jnp.float32)
    is_score = lane32 < float(C)

    zero8 = jnp.zeros((8, N_KEEP_PAD), jnp.float32)
    zerolane = jnp.zeros((1, N_KEEP_PAD), jnp.float32)
    nextoh0 = jnp.where(lane128 == 0.0, 1.0, 0.0)
    oh_last = jnp.where(lane128 == float(MAX_PER_IMAGE - 1), 1.0, 0.0)

    def _gmax(cm):
        return jnp.max(jnp.max(cm, axis=0, keepdims=True),
                       axis=1, keepdims=True)

    def _pfind(cm, gm):
        t = jnp.min(jnp.where(cm == gm, fidx, 1e9),
                    axis=0, keepdims=True)
        return jnp.min(t, axis=1, keepdims=True)

    def cond(carry):
        return carry[0] > -0.5

    def body(carry):
        p_code, cm, kept, valid, nextoh, outv = carry
        p = p_code.astype(jnp.int32)

        row = cand_ref[pl.ds(p, 1), :]
        rs = jnp.where(is_score, row, NEG)
        gmax = jnp.max(rs, axis=1, keepdims=True)
        r_v = jnp.min(jnp.where(rs == gmax, lane32, 1e9),
                      axis=1, keepdims=True)

        bx1 = row[:, C:C + 1]
        by1 = row[:, C + 1:C + 2]
        bx2 = row[:, C + 2:C + 3]
        by2 = row[:, C + 3:C + 4]

        off = (r_v + 1.0) * CLASS_OFFSET
        bxo1 = bx1 + off
        byo1 = by1 + off
        bxo2 = bx2 + off
        byo2 = by2 + off
        barea = (bxo2 - bxo1) * (byo2 - byo1)

        kox1 = kept[0:1, :]
        koy1 = kept[1:2, :]
        kox2 = kept[2:3, :]
        koy2 = kept[3:4, :]
        karea = kept[4:5, :]
        iw = jnp.maximum(jnp.minimum(kox2, bxo2) - jnp.maximum(kox1, bxo1),
                         0.0)
        ih = jnp.maximum(jnp.minimum(koy2, byo2) - jnp.maximum(koy1, byo1),
                         0.0)
        inter = iw * ih
        iou = inter / (karea + barea - inter + 1e-9)
        hit = (iou > NMS_THRESHOLD) & (valid > 0.5)
        suppf = jnp.max(jnp.where(hit, 1.0, 0.0),
                        axis=1, keepdims=True)

        at_lane = nextoh * (1.0 - suppf)
        at_mask = at_lane > 0.5
        kval = jnp.where(sub8 == 0, bxo1,
                         jnp.where(sub8 == 1, byo1,
                                   jnp.where(sub8 == 2, bxo2,
                                             jnp.where(sub8 == 3, byo2,
                                                       barea))))
        kept_new = jnp.where(at_mask & (sub8 < 5), kval, kept)

        oval = jnp.where(sub8 == 0, gmax,
                         jnp.where(sub8 == 1, r_v + 1.0,
                                   jnp.where(sub8 == 2, bx1,
                                             jnp.where(sub8 == 3, by1,
                                                       jnp.where(sub8 == 4,
                                                                 bx2, by2)))))
        outv_new = jnp.where(at_mask & (sub8 < 6), oval, outv)

        valid_new = valid + at_lane
        shifted = jnp.concatenate([jnp.zeros((1, 1), jnp.float32),
                                   nextoh[:, :N_KEEP_PAD - 1]], axis=1)
        nextoh_new = shifted * (1.0 - suppf) + nextoh * suppf

        row_new = jnp.where(lane32 == r_v, NEG, row)
        cand_ref[pl.ds(p, 1), :] = row_new
        head = jnp.max(jnp.where(is_score, row_new, NEG),
                       axis=1, keepdims=True)
        cm_new = jnp.where(fidx == p_code, head, cm)

        gmax_new = _gmax(cm_new)
        p_next = _pfind(cm_new, gmax_new)
        full = jnp.max(valid_new * oh_last, axis=1, keepdims=True)
        stop = (full > 0.5) | (gmax_new < -1e20)
        p_code_next = jnp.where(stop, -1.0, p_next)[0, 0]

        return (p_code_next, cm_new, kept_new, valid_new, nextoh_new,
                outv_new)

    gm0 = _gmax(cm0)
    p_code0 = jnp.where(gm0 < -1e20, -1.0, _pfind(cm0, gm0))[0, 0]
    final = jax.lax.while_loop(
        cond, body, (p_code0, cm0, zero8, zerolane, nextoh0, zero8))
    out_ref[...] = final[5]


@jax.jit
def kernel(cls_logits, bbox_pred, priors):
    logits_t = jnp.transpose(cls_logits[0])
    loc_t = jnp.transpose(bbox_pred[0])
    prior_t = jnp.transpose(priors)

    out = pl.pallas_call(
        _nms_kernel,
        out_shape=jax.ShapeDtypeStruct((8, N_KEEP_PAD), jnp.float32),
        scratch_shapes=[pltpu.VMEM((N_PRIORS, 32), jnp.float32)],
    )(logits_t, loc_t, prior_t)

    out_scores = out[0, :MAX_PER_IMAGE]
    out_labels = out[1, :MAX_PER_IMAGE].astype(jnp.int32)
    out_boxes = jnp.stack(
        [out[2, :MAX_PER_IMAGE], out[3, :MAX_PER_IMAGE],
         out[4, :MAX_PER_IMAGE], out[5, :MAX_PER_IMAGE]], axis=-1)
    return out_boxes, out_scores, out_labels
